# baseline (speedup 1.0000x reference)
import functools

import jax
import jax.numpy as jnp
from jax import lax
from jax.experimental import pallas as pl
from jax.experimental.pallas import tpu as pltpu
from jax.experimental.pallas import tpu_sc as plsc

_N = 320000
_C = 128
_B = 2
_RES = 256
_DIM = _B * _RES * _RES

_NC, _NS = 2, 16
_NW = _NC * _NS
_SEG_W = _DIM // _NW
_SEGCHUNK = 256
_NCHUNK = _SEG_W // _SEGCHUNK
_NBOUND = _DIM // _SEGCHUNK + 1
_BOUND_PAD = 528
_ROWS_B = 128


_NGRP = _ROWS_B // 16


def _sc_body(fmap_hbm, idx_hbm, bounds_hbm, sums_hbm, counts_hbm,
             acc, cnt, cntpart, rowva, rowvb, idxva, idxvb, bnds,
             semra, semia, semrb, semib):
    w = lax.axis_index("s") * _NC + lax.axis_index("c")
    pltpu.sync_copy(bounds_hbm.at[pl.ds(w * _NCHUNK, 32)],
                    bnds.at[pl.ds(0, 32)])
    zero16 = jnp.zeros((16,), jnp.float32)
    iota16 = lax.iota(jnp.int32, 16)
    ones16 = jnp.ones((16,), jnp.float32)
    iota_c = iota16 * _C
    fifteen = jnp.full((16,), 15, jnp.int32)

    def issue(rb, rowv, idxv, semr, semi):
        pltpu.async_copy(idx_hbm.at[pl.ds(rb, _ROWS_B)],
                         idxv.at[pl.ds(0, _ROWS_B)], semi)
        pltpu.async_copy(fmap_hbm.at[pl.ds(rb * _C, _ROWS_B * _C)],
                         rowv.at[pl.ds(0, _ROWS_B * _C)], semr)

    def wait(rb, rowv, idxv, semr, semi):
        pltpu.make_async_copy(idx_hbm.at[pl.ds(rb, _ROWS_B)],
                              idxv.at[pl.ds(0, _ROWS_B)], semi).wait()
        pltpu.make_async_copy(fmap_hbm.at[pl.ds(rb * _C, _ROWS_B * _C)],
                              rowv.at[pl.ds(0, _ROWS_B * _C)], semr).wait()

    def chunk_body(ck, _):
        segbase = (w * _NCHUNK + ck) * _SEGCHUNK
        bv = bnds[pl.ds(ck, 16)]
        lo = bv[0]
        hi = bv[1]
        rb0 = (lo // 8) * 8
        nb = (hi - rb0 + _ROWS_B - 1) // _ROWS_B

        def rb_of(bi):
            return jnp.minimum(rb0 + bi * _ROWS_B, _N - _ROWS_B)

        @pl.when(nb > 0)
        def _():
            issue(rb_of(0), rowva, idxva, semra, semia)

        def zacc(ci, c):
            for k in range(_SEGCHUNK // 16):
                acc[ci, pl.ds(k * 16, 16)] = zero16
            return c

        lax.fori_loop(0, _C, zacc, 0)

        def zcp(l, c):
            for k in range(_SEGCHUNK // 16):
                cntpart[l, pl.ds(k * 16, 16)] = zero16
            return c

        lax.fori_loop(0, 16, zcp, 0)

        def process(rowv, idxv, segbase, r0, r1):
            r0v = jnp.full((16,), r0, jnp.int32)
            r1v = jnp.full((16,), r1, jnp.int32)

            def group_body(g, c):
                iv = idxv[pl.ds(g * 16, 16)]
                segv = iv - segbase
                rowpos = iota16 + g * 16
                m = (rowpos >= r0v) & (rowpos < r1v)
                plsc.addupdate_scatter(cntpart, [iota16, segv], ones16,
                                       mask=m)
                basegj = iota_c + g * 16 * _C
                rotv = iota16
                for p in range(16):
                    base = basegj + rotv
                    for jh in range(0, _C // 16, 4):
                        valss = [plsc.load_gather(rowv,
                                                  [base + (jh + j) * 16])
                                 for j in range(4)]
                        for j in range(4):
                            plsc.addupdate_scatter(
                                acc, [rotv + (jh + j) * 16, segv],
                                valss[j], mask=m)
                    rotv = (rotv + 1) & fifteen
                return c

            lax.fori_loop(0, _NGRP, group_body, 0)

        def batch_body(bi, nextr):
            rb = rb_of(bi)
            r0 = jnp.maximum(nextr, rb) - rb
            r1 = jnp.maximum(r0, jnp.minimum(hi, rb + _ROWS_B) - rb)
            rbn = rb_of(bi + 1)

            @pl.when(bi % 2 == 0)
            def _():
                wait(rb, rowva, idxva, semra, semia)

                @pl.when(bi + 1 < nb)
                def _():
                    issue(rbn, rowvb, idxvb, semrb, semib)

                process(rowva, idxva, segbase, r0, r1)

            @pl.when(bi % 2 == 1)
            def _():
                wait(rb, rowvb, idxvb, semrb, semib)

                @pl.when(bi + 1 < nb)
                def _():
                    issue(rbn, rowva, idxva, semra, semia)

                process(rowvb, idxvb, segbase, r0, r1)

            return rb + r1

        lax.fori_loop(0, nb, batch_body, lo)

        def merge_body(k, c):
            s = cntpart[0, pl.ds(k * 16, 16)]
            for l in range(1, 16):
                s = s + cntpart[l, pl.ds(k * 16, 16)]
            cnt[pl.ds(k * 16, 16)] = s
            return c

        lax.fori_loop(0, _SEGCHUNK // 16, merge_body, 0)
        bb = segbase // (_RES * _RES)
        ii = (segbase // _RES) % _RES
        pltpu.sync_copy(acc, sums_hbm.at[bb, ii])
        pltpu.sync_copy(cnt, counts_hbm.at[bb, ii])
        return 0

    lax.fori_loop(0, _NCHUNK, chunk_body, 0)


_sc_scatter = functools.partial(
    pl.kernel,
    out_type=(
        jax.ShapeDtypeStruct((_B, _RES, _C, _RES), jnp.float32),
        jax.ShapeDtypeStruct((_B, _RES, _RES), jnp.float32),
    ),
    mesh=plsc.VectorSubcoreMesh(
        core_axis_name="c", subcore_axis_name="s",
        num_cores=_NC, num_subcores=_NS),
    compiler_params=pltpu.CompilerParams(use_tc_tiling_on_sc=False,
                                         needs_layout_passes=False),
    scratch_types=[
        pltpu.VMEM((_C, _SEGCHUNK), jnp.float32),
        pltpu.VMEM((_SEGCHUNK,), jnp.float32),
        pltpu.VMEM((16, _SEGCHUNK), jnp.float32),
        pltpu.VMEM((_ROWS_B * _C,), jnp.float32),
        pltpu.VMEM((_ROWS_B * _C,), jnp.float32),
        pltpu.VMEM((_ROWS_B,), jnp.int32),
        pltpu.VMEM((_ROWS_B,), jnp.int32),
        pltpu.VMEM((32,), jnp.int32),
        pltpu.SemaphoreType.DMA,
        pltpu.SemaphoreType.DMA,
        pltpu.SemaphoreType.DMA,
        pltpu.SemaphoreType.DMA,
    ],
)(_sc_body)


_CBLK = 8


def _tc_body(s_ref, c_ref, o_ref):
    x = s_ref[0]
    cnt = jnp.maximum(c_ref[0], 1.0)
    m = x / cnt[:, None, :]
    o_ref[0] = jnp.transpose(m, (1, 2, 0))


def _tc_finish(sums4, counts3):
    ncb = _C // _CBLK
    return pl.pallas_call(
        _tc_body,
        grid=(_B, ncb),
        in_specs=[
            pl.BlockSpec((1, _RES, _CBLK, _RES), lambda b, c: (b, 0, c, 0)),
            pl.BlockSpec((1, _RES, _RES), lambda b, c: (b, 0, 0)),
        ],
        out_specs=pl.BlockSpec((1, _CBLK, _RES, _RES),
                               lambda b, c: (b, c, 0, 0)),
        out_shape=jax.ShapeDtypeStruct((_B, _C, _RES, _RES), jnp.float32),
    )(sums4, counts3)


def kernel(fmap, indices):
    idx32 = indices.astype(jnp.int32)
    bseg = jnp.arange(0, _DIM + 1, _SEGCHUNK, dtype=jnp.int32)
    bounds = jnp.searchsorted(idx32, bseg,
                              method="compare_all").astype(jnp.int32)
    bounds = jnp.pad(bounds, (0, _BOUND_PAD - _NBOUND), constant_values=_N)
    sums4, counts3 = _sc_scatter(fmap.reshape(-1), idx32, bounds)
    return _tc_finish(sums4, counts3)


# speedup vs baseline: 1.7387x; 1.7387x over previous
import functools

import jax
import jax.numpy as jnp
from jax import lax
from jax.experimental import pallas as pl
from jax.experimental.pallas import tpu as pltpu
from jax.experimental.pallas import tpu_sc as plsc

_N = 320000
_C = 128
_B = 2
_RES = 256
_DIM = _B * _RES * _RES

_NC, _NS = 2, 16
_NW = _NC * _NS
_SEG_W = _DIM // _NW
_SEGCHUNK = 256
_NCHUNK = _SEG_W // _SEGCHUNK
_NBOUND = _DIM // _SEGCHUNK + 1
_BOUND_PAD = 528
_ROWS_B = 128


_NGRP = _ROWS_B // 16


def _sc_body(fmap_hbm, idx_hbm, bounds_hbm, sums_hbm, counts_hbm,
             acc, cnt, cntpart, rowva, rowvb, idxva, idxvb, bnds,
             semra, semia, semrb, semib):
    w = lax.axis_index("s") * _NC + lax.axis_index("c")
    pltpu.sync_copy(bounds_hbm.at[pl.ds(w * _NCHUNK, 32)],
                    bnds.at[pl.ds(0, 32)])
    zero16 = jnp.zeros((16,), jnp.float32)
    iota16 = lax.iota(jnp.int32, 16)
    ones16 = jnp.ones((16,), jnp.float32)
    rot = [(iota16 + p) % 16 for p in range(16)]
    gv = [iota16 * _C + rot[p] for p in range(16)]

    def issue(rb, rowv, idxv, semr, semi):
        pltpu.async_copy(idx_hbm.at[pl.ds(rb, _ROWS_B)],
                         idxv.at[pl.ds(0, _ROWS_B)], semi)
        pltpu.async_copy(fmap_hbm.at[pl.ds(rb * _C, _ROWS_B * _C)],
                         rowv.at[pl.ds(0, _ROWS_B * _C)], semr)

    def wait(rb, rowv, idxv, semr, semi):
        pltpu.make_async_copy(idx_hbm.at[pl.ds(rb, _ROWS_B)],
                              idxv.at[pl.ds(0, _ROWS_B)], semi).wait()
        pltpu.make_async_copy(fmap_hbm.at[pl.ds(rb * _C, _ROWS_B * _C)],
                              rowv.at[pl.ds(0, _ROWS_B * _C)], semr).wait()

    def chunk_body(ck, _):
        segbase = (w * _NCHUNK + ck) * _SEGCHUNK
        bv = bnds[pl.ds(ck, 16)]
        lo = bv[0]
        hi = bv[1]
        def zacc(ci, c):
            for k in range(_SEGCHUNK // 16):
                acc[ci, pl.ds(k * 16, 16)] = zero16
            return c

        lax.fori_loop(0, _C, zacc, 0)

        def zcp(l, c):
            for k in range(_SEGCHUNK // 16):
                cntpart[l, pl.ds(k * 16, 16)] = zero16
            return c

        lax.fori_loop(0, 16, zcp, 0)

        rb0 = (lo // 8) * 8
        nb = (hi - rb0 + _ROWS_B - 1) // _ROWS_B

        def rb_of(bi):
            return jnp.minimum(rb0 + bi * _ROWS_B, _N - _ROWS_B)

        @pl.when(nb > 0)
        def _():
            issue(rb_of(0), rowva, idxva, semra, semia)

        def process(rowv, idxv, segbase, r0, r1):
            r0v = jnp.full((16,), r0, jnp.int32)
            r1v = jnp.full((16,), r1, jnp.int32)

            def group_body(g, c):
                iv = idxv[pl.ds(g * 16, 16)]
                segv = iv - segbase
                rowpos = iota16 + g * 16
                m = (rowpos >= r0v) & (rowpos < r1v)
                plsc.addupdate_scatter(cntpart, [iota16, segv], ones16,
                                       mask=m)
                gj = g * 16 * _C
                for p in range(16):
                    base = gv[p] + gj
                    for jh in range(0, _C // 16, 4):
                        valss = [plsc.load_gather(rowv,
                                                  [base + (jh + j) * 16])
                                 for j in range(4)]
                        for j in range(4):
                            plsc.addupdate_scatter(
                                acc, [rot[p] + (jh + j) * 16, segv],
                                valss[j], mask=m)
                return c

            lax.fori_loop(0, _NGRP, group_body, 0)

        def batch_body(bi, nextr):
            rb = rb_of(bi)
            r0 = jnp.maximum(nextr, rb) - rb
            r1 = jnp.maximum(r0, jnp.minimum(hi, rb + _ROWS_B) - rb)
            rbn = rb_of(bi + 1)

            @pl.when(bi % 2 == 0)
            def _():
                wait(rb, rowva, idxva, semra, semia)

                @pl.when(bi + 1 < nb)
                def _():
                    issue(rbn, rowvb, idxvb, semrb, semib)

                process(rowva, idxva, segbase, r0, r1)

            @pl.when(bi % 2 == 1)
            def _():
                wait(rb, rowvb, idxvb, semrb, semib)

                @pl.when(bi + 1 < nb)
                def _():
                    issue(rbn, rowva, idxva, semra, semia)

                process(rowvb, idxvb, segbase, r0, r1)

            return rb + r1

        lax.fori_loop(0, nb, batch_body, lo)

        def merge_body(k, c):
            s = cntpart[0, pl.ds(k * 16, 16)]
            for l in range(1, 16):
                s = s + cntpart[l, pl.ds(k * 16, 16)]
            cnt[pl.ds(k * 16, 16)] = s
            return c

        lax.fori_loop(0, _SEGCHUNK // 16, merge_body, 0)
        bb = segbase // (_RES * _RES)
        ii = (segbase // _RES) % _RES
        pltpu.sync_copy(acc, sums_hbm.at[:, bb, ii])
        pltpu.sync_copy(cnt, counts_hbm.at[bb, ii])
        return 0

    lax.fori_loop(0, _NCHUNK, chunk_body, 0)


_sc_scatter = functools.partial(
    pl.kernel,
    out_type=(
        jax.ShapeDtypeStruct((_C, _B, _RES, _RES), jnp.float32),
        jax.ShapeDtypeStruct((_B, _RES, _RES), jnp.float32),
    ),
    mesh=plsc.VectorSubcoreMesh(
        core_axis_name="c", subcore_axis_name="s",
        num_cores=_NC, num_subcores=_NS),
    compiler_params=pltpu.CompilerParams(use_tc_tiling_on_sc=False,
                                         needs_layout_passes=False),
    scratch_types=[
        pltpu.VMEM((_C, _SEGCHUNK), jnp.float32),
        pltpu.VMEM((_SEGCHUNK,), jnp.float32),
        pltpu.VMEM((16, _SEGCHUNK), jnp.float32),
        pltpu.VMEM((_ROWS_B * _C,), jnp.float32),
        pltpu.VMEM((_ROWS_B * _C,), jnp.float32),
        pltpu.VMEM((_ROWS_B,), jnp.int32),
        pltpu.VMEM((_ROWS_B,), jnp.int32),
        pltpu.VMEM((32,), jnp.int32),
        pltpu.SemaphoreType.DMA,
        pltpu.SemaphoreType.DMA,
        pltpu.SemaphoreType.DMA,
        pltpu.SemaphoreType.DMA,
    ],
)(_sc_body)


_CBLK = 8


def _tc_body(s_ref, c_ref, o_ref):
    x = s_ref[:, 0]
    cnt = jnp.maximum(c_ref[0], 1.0)
    m = x / cnt[None, :, :]
    o_ref[0] = jnp.transpose(m, (0, 2, 1))


def _tc_finish(sums4, counts3):
    ncb = _C // _CBLK
    return pl.pallas_call(
        _tc_body,
        grid=(_B, ncb),
        in_specs=[
            pl.BlockSpec((_CBLK, 1, _RES, _RES), lambda b, c: (c, b, 0, 0)),
            pl.BlockSpec((1, _RES, _RES), lambda b, c: (b, 0, 0)),
        ],
        out_specs=pl.BlockSpec((1, _CBLK, _RES, _RES),
                               lambda b, c: (b, c, 0, 0)),
        out_shape=jax.ShapeDtypeStruct((_B, _C, _RES, _RES), jnp.float32),
    )(sums4, counts3)


def kernel(fmap, indices):
    idx32 = indices.astype(jnp.int32)
    bseg = jnp.arange(0, _DIM + 1, _SEGCHUNK, dtype=jnp.int32)
    bounds = jnp.searchsorted(idx32, bseg,
                              method="compare_all").astype(jnp.int32)
    bounds = jnp.pad(bounds, (0, _BOUND_PAD - _NBOUND), constant_values=_N)
    sums4, counts3 = _sc_scatter(fmap.reshape(-1), idx32, bounds)
    return _tc_finish(sums4, counts3)
